# baseline (speedup 1.0000x reference)
import functools

import jax
import jax.numpy as jnp
from jax import lax
from jax.experimental import pallas as pl
from jax.experimental.pallas import tpu as pltpu
from jax.experimental.pallas import tpu_sc as plsc

_N0, _N1, _N2 = 10000, 160000, 80000
_HID = 64
_NG = 16
_NC, _NS = 2, 16
_NW = _NC * _NS
_GK = 256
_SK = 2048
_EGRAN = 32768
_ZR = 256

_f32 = jnp.float32
_i32 = jnp.int32


def _silu(x):
    return x * jax.nn.sigmoid(x)


def _mm_bias_body(x_ref, w_ref, b_ref, o_ref):
    o_ref[...] = jnp.dot(x_ref[...], w_ref[...],
                         preferred_element_type=_f32) + b_ref[...]


def _mm_bias(x, w, b, bn=2048):
    n, d = x.shape
    k = w.shape[1]
    return pl.pallas_call(
        _mm_bias_body,
        grid=(pl.cdiv(n, bn),),
        in_specs=[pl.BlockSpec((bn, d), lambda i: (i, 0)),
                  pl.BlockSpec((d, k), lambda i: (0, 0)),
                  pl.BlockSpec((1, k), lambda i: (0, 0))],
        out_specs=pl.BlockSpec((bn, k), lambda i: (i, 0)),
        out_shape=jax.ShapeDtypeStruct((n, k), _f32),
    )(x, w, b.reshape(1, k))


def _stage1_body(h_ref, ws_ref, bs_ref, *o_refs):
    h = h_ref[...]
    for p, o_ref in enumerate(o_refs):
        left = jnp.dot(h, ws_ref[2 * p], preferred_element_type=_f32) + bs_ref[2 * p]
        right = jnp.dot(h, ws_ref[2 * p + 1],
                        preferred_element_type=_f32) + bs_ref[2 * p + 1]
        o_ref[...] = jnp.concatenate([left, right], axis=1)


def _stage1(h, ws, bs, bn=2048):
    n = h.shape[0]
    p = ws.shape[0] // 2
    return pl.pallas_call(
        _stage1_body,
        grid=(pl.cdiv(n, bn),),
        in_specs=[pl.BlockSpec((bn, _HID), lambda i: (i, 0)),
                  pl.BlockSpec((2 * p, _HID, _HID), lambda i: (0, 0, 0)),
                  pl.BlockSpec((2 * p, 1, _HID), lambda i: (0, 0, 0))],
        out_specs=[pl.BlockSpec((bn, 2 * _HID), lambda i: (i, 0))] * p,
        out_shape=[jax.ShapeDtypeStruct((n, 2 * _HID), _f32)] * p,
    )(h, ws, bs)


def _edge_mlp_body(tab_ref, inv_ref, w1i_ref, w2_ref, b2_ref, o_ref,
                   *, e, ni, be):
    t = tab_ref[...]
    for k in range(ni):
        t += inv_ref[:, k:k + 1] * w1i_ref[k]
    t = _silu(t)
    m = _silu(jnp.dot(t, w2_ref[...], preferred_element_type=_f32) + b2_ref[...])
    row = pl.program_id(0) * be + lax.broadcasted_iota(_i32, (be, 1), 0)
    o_ref[...] = jnp.where(row < e, m, 0.0)


def _edge_mlp(tab, inv_pad, w1i, w2, b2, e, be=1024):
    epad = tab.shape[0]
    ni = inv_pad.shape[1]
    body = functools.partial(_edge_mlp_body, e=e, ni=ni, be=be)
    return pl.pallas_call(
        body,
        grid=(epad // be,),
        in_specs=[pl.BlockSpec((be, _HID), lambda i: (i, 0)),
                  pl.BlockSpec((be, ni), lambda i: (i, 0)),
                  pl.BlockSpec((ni, 1, _HID), lambda i: (0, 0, 0)),
                  pl.BlockSpec((_HID, _HID), lambda i: (0, 0)),
                  pl.BlockSpec((1, _HID), lambda i: (0, 0))],
        out_specs=pl.BlockSpec((be, _HID), lambda i: (i, 0)),
        out_shape=jax.ShapeDtypeStruct((epad, _HID), _f32),
    )(tab, inv_pad, w1i.reshape(ni, 1, _HID), w2, b2.reshape(1, _HID))


def _upd_body(h_ref, *rest):
    m_refs = rest[:-3]
    wu_ref, b_ref, o_ref = rest[-3:]
    h = h_ref[...]
    m = m_refs[0][...]
    for r in m_refs[1:]:
        m = m + r[...]
    z = jnp.concatenate([h, m], axis=1)
    o_ref[...] = h + _silu(jnp.dot(z, wu_ref[...],
                                   preferred_element_type=_f32) + b_ref[...])


def _update(h, ms, wu, b, bn=2048):
    n = h.shape[0]
    nm = len(ms)
    return pl.pallas_call(
        _upd_body,
        grid=(pl.cdiv(n, bn),),
        in_specs=[pl.BlockSpec((bn, _HID), lambda i: (i, 0))] * (1 + nm)
        + [pl.BlockSpec((2 * _HID, _HID), lambda i: (0, 0)),
           pl.BlockSpec((1, _HID), lambda i: (0, 0))],
        out_specs=pl.BlockSpec((bn, _HID), lambda i: (i, 0)),
        out_shape=jax.ShapeDtypeStruct((n, _HID), _f32),
    )(h, *ms, wu, b.reshape(1, _HID))


def _pool_body(h_ref, b_ref, w1_ref, b1_ref, w2_ref, b2_ref, o_ref, *, n, bn):
    i = pl.program_id(0)

    @pl.when(i == 0)
    def _():
        o_ref[...] = jnp.zeros_like(o_ref)

    hp = _silu(jnp.dot(h_ref[...], w1_ref[...],
                       preferred_element_type=_f32) + b1_ref[...])
    hp = jnp.dot(hp, w2_ref[...], preferred_element_type=_f32) + b2_ref[...]
    col = i * bn + lax.broadcasted_iota(_i32, (1, bn), 1)
    valid = col < n
    rowi = i * bn + lax.broadcasted_iota(_i32, (bn, 1), 0)
    hp = jnp.where(rowi < n, hp, 0.0)
    gid = lax.broadcasted_iota(_i32, (_NG, bn), 0)
    oh = jnp.where((b_ref[...] == gid) & valid, 1.0, 0.0)
    o_ref[...] += jnp.dot(oh, hp, preferred_element_type=_f32)


def _pool(h, batch, pp, bn=2048):
    n = h.shape[0]
    body = functools.partial(_pool_body, n=n, bn=bn)
    return pl.pallas_call(
        body,
        grid=(pl.cdiv(n, bn),),
        in_specs=[pl.BlockSpec((bn, _HID), lambda i: (i, 0)),
                  pl.BlockSpec((1, bn), lambda i: (0, i)),
                  pl.BlockSpec((_HID, _HID), lambda i: (0, 0)),
                  pl.BlockSpec((1, _HID), lambda i: (0, 0)),
                  pl.BlockSpec((_HID, _HID), lambda i: (0, 0)),
                  pl.BlockSpec((1, _HID), lambda i: (0, 0))],
        out_specs=pl.BlockSpec((_NG, _HID), lambda i: (0, 0)),
        out_shape=jax.ShapeDtypeStruct((_NG, _HID), _f32),
    )(h, batch.reshape(1, n), pp["l1"]["W"], pp["l1"]["b"].reshape(1, _HID),
      pp["l2"]["W"], pp["l2"]["b"].reshape(1, _HID))


def _post_body(s_ref, w1_ref, b1_ref, w2_ref, b2_ref, o_ref):
    t = _silu(jnp.dot(s_ref[...], w1_ref[...],
                      preferred_element_type=_f32) + b1_ref[...])
    o_ref[...] = jnp.dot(t, w2_ref[...], preferred_element_type=_f32) + b2_ref[...]


@functools.lru_cache(maxsize=None)
def _make_gather(epad):
    mesh = plsc.VectorSubcoreMesh(core_axis_name="c", subcore_axis_name="s",
                                  num_cores=_NC, num_subcores=_NS)
    per_w = epad // _NW
    nsup = per_w // 2048
    nstr = _GK // 128
    nrnd = 2048 // _GK

    @functools.partial(
        pl.kernel,
        out_type=jax.ShapeDtypeStruct((epad, _HID), _f32),
        mesh=mesh,
        scratch_types=[pltpu.VMEM((16, 128), _i32),
                       pltpu.VMEM((16, 128), _i32),
                       pltpu.VMEM((_GK, 2 * _HID), _f32),
                       pltpu.VMEM((_GK, 2 * _HID), _f32),
                       pltpu.VMEM((_GK, _HID), _f32),
                       pltpu.SemaphoreType.DMA,
                       pltpu.SemaphoreType.DMA],
    )
    def gk(a_hbm, b_hbm, recv2, send2, tsum_hbm,
           ridx, sidx, bufr, bufs, bufo, semi, semg):
        wid = lax.axis_index("s") * _NC + lax.axis_index("c")

        def sup(w, carry):
            sbase = wid * per_w + w * 2048
            irow = pl.multiple_of(sbase // 128, 8)
            c1 = pltpu.async_copy(recv2.at[pl.ds(irow, 16)], ridx, semi)
            c2 = pltpu.async_copy(send2.at[pl.ds(irow, 16)], sidx, semi)
            c1.wait()
            c2.wait()
            for h in range(nrnd):
                base = pl.multiple_of(sbase + h * _GK, _GK)
                cps = [pltpu.async_copy(
                           a_hbm.at[ridx.at[h * nstr + j]],
                           bufr.at[pl.ds(j * 128, 128)], semg)
                       for j in range(nstr)]
                cps += [pltpu.async_copy(
                            b_hbm.at[sidx.at[h * nstr + j]],
                            bufs.at[pl.ds(j * 128, 128)], semg)
                        for j in range(nstr)]
                for cp in cps:
                    cp.wait()

                def add_row(rr, carry2):
                    for cc in range(_HID // 16):
                        bufo[rr, pl.ds(cc * 16, 16)] = (
                            bufr[rr, pl.ds(cc * 16, 16)]
                            + bufs[rr, pl.ds(_HID + cc * 16, 16)])
                    return carry2

                lax.fori_loop(0, _GK, add_row, 0)
                pltpu.sync_copy(bufo, tsum_hbm.at[pl.ds(base, _GK)])
            return carry

        lax.fori_loop(0, nsup, sup, 0)

    return gk


@functools.lru_cache(maxsize=None)
def _make_scatter(epad, nr, nch):
    r = nr // nch
    assert r * nch == nr and r % 8 == 0
    nchc = nch // 2
    nstr = _SK // 128
    rb = (r // _NS) // 8 * 8
    rem = r - _NS * rb
    nzg = -(-(r + 16) // _ZR)
    mesh = plsc.VectorSubcoreMesh(core_axis_name="c", subcore_axis_name="s",
                                  num_cores=_NC, num_subcores=_NS)

    @functools.partial(
        pl.kernel,
        out_type=jax.ShapeDtypeStruct((nr, _HID), _f32),
        mesh=mesh,
        scratch_types=[pltpu.VMEM((64,), _i32),
                       pltpu.VMEM((nstr, 128), _i32),
                       pltpu.VMEM((nstr, 128), _i32),
                       pltpu.VMEM((256, _HID), _f32),
                       pltpu.VMEM_SHARED((r + 16, _HID), _f32),
                       pltpu.SemaphoreType.DMA],
    )
    def sk(m_hbm, recv2, off_hbm, zeros_hbm, out_hbm,
           offv, idxb, idxl, mbuf, acc, sem):
        c = lax.axis_index("c")
        s = lax.axis_index("s")
        pltpu.sync_copy(off_hbm, offv)
        iot = lax.broadcasted_iota(_i32, (16,), 0)
        ovs = [offv[pl.ds(16 * i, 16)] for i in range(2)]

        def _sext(k):
            return ovs[k // 16][k % 16]

        def _off(k):
            return jnp.where(c == 0, _sext(k), _sext(nchc + k))

        for ci in range(nchc):
            chunk = c * nchc + ci
            lo_r = chunk * r
            pltpu.sync_copy(zeros_hbm, mbuf)
            for zg in range(nzg):
                zsz = min(_ZR, r + 16 - zg * _ZR)

                @pl.when(s == zg % _NS)
                def _():
                    pltpu.sync_copy(mbuf.at[pl.ds(0, zsz)],
                                    acc.at[pl.ds(zg * _ZR, zsz)])
            plsc.subcore_barrier()

            lo_e = _off(ci)
            hi_e = _off(ci + 1)
            lo_al = (lo_e // _SK) * _SK
            nw = jnp.maximum(0, (hi_e - lo_al + (_SK - 1)) // _SK - s + 15) // 16

            def win(wi, carry):
                base = pl.multiple_of(lo_al + (s + wi * _NS) * _SK, _SK)
                pltpu.sync_copy(recv2.at[pl.ds(pl.multiple_of(base // 128, 8),
                                               nstr)], idxb)
                for j in range(nstr):
                    for cc in range(8):
                        v = idxb[j, pl.ds(cc * 16, 16)]
                        ok = (v >= lo_r) & (v < lo_r + r)
                        idxl[j, pl.ds(cc * 16, 16)] = jnp.where(
                            ok, v - lo_r, r + iot)
                for q in range(_SK // 256):
                    pltpu.sync_copy(
                        m_hbm.at[pl.ds(pl.multiple_of(base + q * 256, 8), 256)],
                        mbuf)
                    cps = [pltpu.async_copy(mbuf.at[pl.ds(jj * 128, 128)],
                                            acc.at[idxl.at[q * 2 + jj]],
                                            sem, add=True)
                           for jj in range(2)]
                    for cp in cps:
                        cp.wait()
                return carry

            lax.fori_loop(0, nw, win, 0)
            plsc.subcore_barrier()
            pltpu.sync_copy(acc.at[pl.ds(pl.multiple_of(s * rb, 8), rb)],
                            out_hbm.at[pl.ds(pl.multiple_of(lo_r + s * rb, 8),
                                             rb)])
            if rem:
                @pl.when(s == 0)
                def _():
                    pltpu.sync_copy(acc.at[pl.ds(_NS * rb, rem)],
                                    out_hbm.at[pl.ds(
                                        pl.multiple_of(lo_r + _NS * rb, 8),
                                        rem)])
            plsc.subcore_barrier()

    return sk


def _sort_pad_edges(recv, send, inv, nr, ns, nch):
    e = recv.shape[0]
    recv = recv.astype(_i32)
    perm = jnp.argsort(recv)
    recv_s = recv[perm]
    send_s = send.astype(_i32)[perm]
    inv_s = inv[perm]
    r = nr // nch
    off = jnp.searchsorted(recv_s, jnp.arange(nch + 1, dtype=_i32) * r)
    off = jnp.concatenate(
        [off.astype(_i32), jnp.zeros((64 - nch - 1,), _i32)])
    epad = ((e + _EGRAN - 1) // _EGRAN) * _EGRAN
    pad = epad - e
    fill = jnp.arange(pad, dtype=_i32)
    recv_p = jnp.concatenate([recv_s, fill % nr])
    send_p = jnp.concatenate([send_s, fill % ns])
    inv_p = jnp.concatenate([inv_s, jnp.zeros((pad, inv.shape[1]), _f32)])
    return (recv_p.reshape(epad // 128, 128), send_p.reshape(epad // 128, 128),
            inv_p, off, e, epad)


def _split_msg(mp, ni):
    w1 = mp["l1"]["W"]
    return {"w1r": w1[:_HID], "w1s": w1[_HID:2 * _HID], "w1i": w1[2 * _HID:],
            "b1": mp["l1"]["b"], "w2": mp["l2"]["W"], "b2": mp["l2"]["b"]}


def kernel(x_0, x_1, x_2, adj_0, adj_1, inc_0_1, inc_1_2, inv_0_0, inv_1_1,
           inv_0_1, inv_1_2, batch_0, batch_1, batch_2, params):
    p00 = _sort_pad_edges(adj_0[1], adj_0[0], inv_0_0, _N0, _N0, 2)
    p11 = _sort_pad_edges(adj_1[1], adj_1[0], inv_1_1, _N1, _N1, 20)
    p01 = _sort_pad_edges(inc_0_1[1], inc_0_1[0], inv_0_1, _N1, _N0, 20)
    p12 = _sort_pad_edges(inc_1_2[1], inc_1_2[0], inv_1_2, _N2, _N1, 20)
    zeros = jnp.zeros((_ZR, _HID), _f32)

    emb = params["emb"]
    h0 = _mm_bias(x_0, emb["W"], emb["b"])
    h1 = _mm_bias(x_1, emb["W"], emb["b"])
    h2 = _mm_bias(x_2, emb["W"], emb["b"])

    for lp in params["layers"]:
        sp = {k: _split_msg(lp[k], ni) for k, ni in
              (("msg_adj_0", 3), ("msg_adj_1", 6),
               ("msg_inc_0_1", 3), ("msg_inc_1_2", 6))}
        zb = jnp.zeros((_HID,), _f32)
        t0a, t0b = _stage1(
            h0,
            jnp.stack([sp["msg_adj_0"]["w1r"], sp["msg_adj_0"]["w1s"],
                       sp["msg_inc_0_1"]["w1s"], sp["msg_inc_0_1"]["w1s"]]),
            jnp.stack([sp["msg_adj_0"]["b1"], zb, zb, zb]).reshape(4, 1, _HID))
        t1, t1b = _stage1(
            h1,
            jnp.stack([sp["msg_adj_1"]["w1r"], sp["msg_adj_1"]["w1s"],
                       sp["msg_inc_0_1"]["w1r"], sp["msg_inc_1_2"]["w1s"]]),
            jnp.stack([sp["msg_adj_1"]["b1"], zb, sp["msg_inc_0_1"]["b1"],
                       zb]).reshape(4, 1, _HID))
        (t2,) = _stage1(
            h2,
            jnp.stack([sp["msg_inc_1_2"]["w1r"], sp["msg_inc_1_2"]["w1r"]]),
            jnp.stack([sp["msg_inc_1_2"]["b1"],
                       sp["msg_inc_1_2"]["b1"]]).reshape(2, 1, _HID))

        msums = []
        for (name, atab, btab, pe, nr, nch) in (
                ("msg_adj_0", t0a, t0a, p00, _N0, 2),
                ("msg_adj_1", t1, t1, p11, _N1, 20),
                ("msg_inc_0_1", t1b, t0b, p01, _N1, 20),
                ("msg_inc_1_2", t2, t1b, p12, _N2, 20)):
            recv2, send2, inv_p, off, e, epad = pe
            tsum = _make_gather(epad)(atab, btab, recv2, send2)
            m = _edge_mlp(tsum, inv_p, sp[name]["w1i"], sp[name]["w2"],
                          sp[name]["b2"], e)
            msums.append(_make_scatter(epad, nr, nch)(m, recv2, off, zeros))

        h0 = _update(h0, [msums[0]], lp["upd_0"]["W"], lp["upd_0"]["b"])
        h1 = _update(h1, [msums[1], msums[2]], lp["upd_1"]["W"], lp["upd_1"]["b"])
        h2 = _update(h2, [msums[3]], lp["upd_2"]["W"], lp["upd_2"]["b"])

    pools = [_pool(h0, batch_0, params["pre"]["rank_0"]),
             _pool(h1, batch_1, params["pre"]["rank_1"]),
             _pool(h2, batch_2, params["pre"]["rank_2"])]
    state = jnp.concatenate(pools, axis=1)
    post = params["post"]
    out = pl.pallas_call(
        _post_body,
        out_shape=jax.ShapeDtypeStruct((_NG, 1), _f32),
    )(state, post["l1"]["W"], post["l1"]["b"].reshape(1, _HID),
      post["l2"]["W"], post["l2"]["b"].reshape(1, 1))
    return jnp.squeeze(out)


# speedup vs baseline: 1.0206x; 1.0206x over previous
import functools

import jax
import jax.numpy as jnp
from jax import lax
from jax.experimental import pallas as pl
from jax.experimental.pallas import tpu as pltpu
from jax.experimental.pallas import tpu_sc as plsc

_N0, _N1, _N2 = 10000, 160000, 80000
_HID = 64
_NG = 16
_NC, _NS = 2, 16
_NW = _NC * _NS
_GK = 256
_SK = 1024
_EGRAN = 32768
_ZR = 256

_f32 = jnp.float32
_i32 = jnp.int32


def _silu(x):
    return x * jax.nn.sigmoid(x)


def _mm_bias_body(x_ref, w_ref, b_ref, o_ref):
    o_ref[...] = jnp.dot(x_ref[...], w_ref[...],
                         preferred_element_type=_f32) + b_ref[...]


def _mm_bias(x, w, b, bn=2048):
    n, d = x.shape
    k = w.shape[1]
    return pl.pallas_call(
        _mm_bias_body,
        grid=(pl.cdiv(n, bn),),
        in_specs=[pl.BlockSpec((bn, d), lambda i: (i, 0)),
                  pl.BlockSpec((d, k), lambda i: (0, 0)),
                  pl.BlockSpec((1, k), lambda i: (0, 0))],
        out_specs=pl.BlockSpec((bn, k), lambda i: (i, 0)),
        out_shape=jax.ShapeDtypeStruct((n, k), _f32),
    )(x, w, b.reshape(1, k))


def _stage1_body(h_ref, ws_ref, bs_ref, *o_refs):
    h = h_ref[...]
    for p, o_ref in enumerate(o_refs):
        left = jnp.dot(h, ws_ref[2 * p], preferred_element_type=_f32) + bs_ref[2 * p]
        right = jnp.dot(h, ws_ref[2 * p + 1],
                        preferred_element_type=_f32) + bs_ref[2 * p + 1]
        o_ref[...] = jnp.concatenate([left, right], axis=1)


def _stage1(h, ws, bs, bn=2048):
    n = h.shape[0]
    p = ws.shape[0] // 2
    return pl.pallas_call(
        _stage1_body,
        grid=(pl.cdiv(n, bn),),
        in_specs=[pl.BlockSpec((bn, _HID), lambda i: (i, 0)),
                  pl.BlockSpec((2 * p, _HID, _HID), lambda i: (0, 0, 0)),
                  pl.BlockSpec((2 * p, 1, _HID), lambda i: (0, 0, 0))],
        out_specs=[pl.BlockSpec((bn, 2 * _HID), lambda i: (i, 0))] * p,
        out_shape=[jax.ShapeDtypeStruct((n, 2 * _HID), _f32)] * p,
    )(h, ws, bs)


def _edge_mlp_body(tab_ref, inv_ref, w1i_ref, w2_ref, b2_ref, o_ref,
                   *, e, ni, be):
    t = tab_ref[...]
    for k in range(ni):
        t += inv_ref[:, k:k + 1] * w1i_ref[k]
    t = _silu(t)
    m = _silu(jnp.dot(t, w2_ref[...], preferred_element_type=_f32) + b2_ref[...])
    row = pl.program_id(0) * be + lax.broadcasted_iota(_i32, (be, 1), 0)
    o_ref[...] = jnp.where(row < e, m, 0.0)


def _edge_mlp(tab, inv_pad, w1i, w2, b2, e, be=1024):
    epad = tab.shape[0]
    ni = inv_pad.shape[1]
    body = functools.partial(_edge_mlp_body, e=e, ni=ni, be=be)
    return pl.pallas_call(
        body,
        grid=(epad // be,),
        in_specs=[pl.BlockSpec((be, _HID), lambda i: (i, 0)),
                  pl.BlockSpec((be, ni), lambda i: (i, 0)),
                  pl.BlockSpec((ni, 1, _HID), lambda i: (0, 0, 0)),
                  pl.BlockSpec((_HID, _HID), lambda i: (0, 0)),
                  pl.BlockSpec((1, _HID), lambda i: (0, 0))],
        out_specs=pl.BlockSpec((be, _HID), lambda i: (i, 0)),
        out_shape=jax.ShapeDtypeStruct((epad, _HID), _f32),
    )(tab, inv_pad, w1i.reshape(ni, 1, _HID), w2, b2.reshape(1, _HID))


def _upd_body(h_ref, *rest):
    m_refs = rest[:-3]
    wu_ref, b_ref, o_ref = rest[-3:]
    h = h_ref[...]
    m = m_refs[0][...]
    for r in m_refs[1:]:
        m = m + r[...]
    z = jnp.concatenate([h, m], axis=1)
    o_ref[...] = h + _silu(jnp.dot(z, wu_ref[...],
                                   preferred_element_type=_f32) + b_ref[...])


def _update(h, ms, wu, b, bn=2048):
    n = h.shape[0]
    nm = len(ms)
    return pl.pallas_call(
        _upd_body,
        grid=(pl.cdiv(n, bn),),
        in_specs=[pl.BlockSpec((bn, _HID), lambda i: (i, 0))] * (1 + nm)
        + [pl.BlockSpec((2 * _HID, _HID), lambda i: (0, 0)),
           pl.BlockSpec((1, _HID), lambda i: (0, 0))],
        out_specs=pl.BlockSpec((bn, _HID), lambda i: (i, 0)),
        out_shape=jax.ShapeDtypeStruct((n, _HID), _f32),
    )(h, *ms, wu, b.reshape(1, _HID))


def _pool_body(h_ref, b_ref, w1_ref, b1_ref, w2_ref, b2_ref, o_ref, *, n, bn):
    i = pl.program_id(0)

    @pl.when(i == 0)
    def _():
        o_ref[...] = jnp.zeros_like(o_ref)

    hp = _silu(jnp.dot(h_ref[...], w1_ref[...],
                       preferred_element_type=_f32) + b1_ref[...])
    hp = jnp.dot(hp, w2_ref[...], preferred_element_type=_f32) + b2_ref[...]
    col = i * bn + lax.broadcasted_iota(_i32, (1, bn), 1)
    valid = col < n
    rowi = i * bn + lax.broadcasted_iota(_i32, (bn, 1), 0)
    hp = jnp.where(rowi < n, hp, 0.0)
    gid = lax.broadcasted_iota(_i32, (_NG, bn), 0)
    oh = jnp.where((b_ref[...] == gid) & valid, 1.0, 0.0)
    o_ref[...] += jnp.dot(oh, hp, preferred_element_type=_f32)


def _pool(h, batch, pp, bn=2048):
    n = h.shape[0]
    body = functools.partial(_pool_body, n=n, bn=bn)
    return pl.pallas_call(
        body,
        grid=(pl.cdiv(n, bn),),
        in_specs=[pl.BlockSpec((bn, _HID), lambda i: (i, 0)),
                  pl.BlockSpec((1, bn), lambda i: (0, i)),
                  pl.BlockSpec((_HID, _HID), lambda i: (0, 0)),
                  pl.BlockSpec((1, _HID), lambda i: (0, 0)),
                  pl.BlockSpec((_HID, _HID), lambda i: (0, 0)),
                  pl.BlockSpec((1, _HID), lambda i: (0, 0))],
        out_specs=pl.BlockSpec((_NG, _HID), lambda i: (0, 0)),
        out_shape=jax.ShapeDtypeStruct((_NG, _HID), _f32),
    )(h, batch.reshape(1, n), pp["l1"]["W"], pp["l1"]["b"].reshape(1, _HID),
      pp["l2"]["W"], pp["l2"]["b"].reshape(1, _HID))


def _post_body(s_ref, w1_ref, b1_ref, w2_ref, b2_ref, o_ref):
    t = _silu(jnp.dot(s_ref[...], w1_ref[...],
                      preferred_element_type=_f32) + b1_ref[...])
    o_ref[...] = jnp.dot(t, w2_ref[...], preferred_element_type=_f32) + b2_ref[...]


@functools.lru_cache(maxsize=None)
def _make_gather(epad):
    mesh = plsc.VectorSubcoreMesh(core_axis_name="c", subcore_axis_name="s",
                                  num_cores=_NC, num_subcores=_NS)
    per_w = epad // _NW
    nsup = per_w // 1024
    nrnd = 8

    @functools.partial(
        pl.kernel,
        out_type=jax.ShapeDtypeStruct((epad, _HID), _f32),
        mesh=mesh,
        scratch_types=[pltpu.VMEM((8, 128), _i32),
                       pltpu.VMEM((8, 128), _i32),
                       pltpu.VMEM((128, 2 * _HID), _f32),
                       pltpu.VMEM((128, 2 * _HID), _f32),
                       pltpu.VMEM((128, 2 * _HID), _f32),
                       pltpu.VMEM((128, 2 * _HID), _f32),
                       pltpu.VMEM((128, _HID), _f32),
                       pltpu.VMEM((128, _HID), _f32),
                       pltpu.SemaphoreType.DMA,
                       pltpu.SemaphoreType.DMA],
    )
    def gk(a_hbm, b_hbm, recv2, send2, tsum_hbm,
           ridx, sidx, br0, br1, bs0, bs1, bo0, bo1, semg, semo):
        wid = lax.axis_index("s") * _NC + lax.axis_index("c")
        brs, bss, bos = [br0, br1], [bs0, bs1], [bo0, bo1]

        def sup(w, carry):
            sbase = wid * per_w + w * 1024
            irow = pl.multiple_of(sbase // 128, 8)
            pltpu.sync_copy(recv2.at[pl.ds(irow, 8)], ridx)
            pltpu.sync_copy(send2.at[pl.ds(irow, 8)], sidx)

            def fire(h):
                return [pltpu.async_copy(a_hbm.at[ridx.at[h]],
                                         brs[h % 2], semg),
                        pltpu.async_copy(b_hbm.at[sidx.at[h]],
                                         bss[h % 2], semg)]

            gath = {0: fire(0)}
            outs = {}
            for h in range(nrnd):
                if h + 1 < nrnd:
                    gath[h + 1] = fire(h + 1)
                for cp in gath[h]:
                    cp.wait()
                if h >= 2:
                    outs[h - 2].wait()
                br, bs, bo = brs[h % 2], bss[h % 2], bos[h % 2]

                def add_row(rr, carry2):
                    for cc in range(_HID // 16):
                        bo[rr, pl.ds(cc * 16, 16)] = (
                            br[rr, pl.ds(cc * 16, 16)]
                            + bs[rr, pl.ds(_HID + cc * 16, 16)])
                    return carry2

                lax.fori_loop(0, 128, add_row, 0)
                base = pl.multiple_of(sbase + h * 128, 8)
                outs[h] = pltpu.async_copy(bo, tsum_hbm.at[pl.ds(base, 128)],
                                           semo)
            outs[nrnd - 2].wait()
            outs[nrnd - 1].wait()
            return carry

        lax.fori_loop(0, nsup, sup, 0)

    return gk


@functools.lru_cache(maxsize=None)
def _make_scatter(epad, nr, nch):
    r = nr // nch
    assert r * nch == nr and r % 8 == 0
    nchc = nch // 2
    nstr = _SK // 128
    rb = (r // _NS) // 8 * 8
    rem = r - _NS * rb
    nzg = -(-(r + 16) // _ZR)
    mesh = plsc.VectorSubcoreMesh(core_axis_name="c", subcore_axis_name="s",
                                  num_cores=_NC, num_subcores=_NS)

    @functools.partial(
        pl.kernel,
        out_type=jax.ShapeDtypeStruct((nr, _HID), _f32),
        mesh=mesh,
        scratch_types=[pltpu.VMEM((64,), _i32),
                       pltpu.VMEM((nstr, 128), _i32),
                       pltpu.VMEM((nstr, 128), _i32),
                       pltpu.VMEM((256, _HID), _f32),
                       pltpu.VMEM_SHARED((r + 16, _HID), _f32),
                       pltpu.SemaphoreType.DMA],
    )
    def sk(m_hbm, recv2, off_hbm, zeros_hbm, out_hbm,
           offv, idxb, idxl, mbuf, acc, sem):
        c = lax.axis_index("c")
        s = lax.axis_index("s")
        pltpu.sync_copy(off_hbm, offv)
        iot = lax.broadcasted_iota(_i32, (16,), 0)
        ovs = [offv[pl.ds(16 * i, 16)] for i in range(2)]

        def _sext(k):
            return ovs[k // 16][k % 16]

        def _off(k):
            return jnp.where(c == 0, _sext(k), _sext(nchc + k))

        for ci in range(nchc):
            chunk = c * nchc + ci
            lo_r = chunk * r
            pltpu.sync_copy(zeros_hbm, mbuf)
            for zg in range(nzg):
                zsz = min(_ZR, r + 16 - zg * _ZR)

                @pl.when(s == zg % _NS)
                def _():
                    pltpu.sync_copy(mbuf.at[pl.ds(0, zsz)],
                                    acc.at[pl.ds(zg * _ZR, zsz)])
            plsc.subcore_barrier()

            lo_e = _off(ci)
            hi_e = _off(ci + 1)
            lo_al = (lo_e // _SK) * _SK
            nw = jnp.maximum(0, (hi_e - lo_al + (_SK - 1)) // _SK - s + 15) // 16

            def win(wi, carry):
                base = pl.multiple_of(lo_al + (s + wi * _NS) * _SK, _SK)
                pltpu.sync_copy(recv2.at[pl.ds(pl.multiple_of(base // 128, 8),
                                               nstr)], idxb)
                for j in range(nstr):
                    for cc in range(8):
                        v = idxb[j, pl.ds(cc * 16, 16)]
                        ok = (v >= lo_r) & (v < lo_r + r)
                        idxl[j, pl.ds(cc * 16, 16)] = jnp.where(
                            ok, v - lo_r, r + iot)
                for q in range(_SK // 256):
                    pltpu.sync_copy(
                        m_hbm.at[pl.ds(pl.multiple_of(base + q * 256, 8), 256)],
                        mbuf)
                    cps = [pltpu.async_copy(mbuf.at[pl.ds(jj * 128, 128)],
                                            acc.at[idxl.at[q * 2 + jj]],
                                            sem, add=True)
                           for jj in range(2)]
                    for cp in cps:
                        cp.wait()
                return carry

            lax.fori_loop(0, nw, win, 0)
            plsc.subcore_barrier()
            pltpu.sync_copy(acc.at[pl.ds(pl.multiple_of(s * rb, 8), rb)],
                            out_hbm.at[pl.ds(pl.multiple_of(lo_r + s * rb, 8),
                                             rb)])
            if rem:
                @pl.when(s == 0)
                def _():
                    pltpu.sync_copy(acc.at[pl.ds(_NS * rb, rem)],
                                    out_hbm.at[pl.ds(
                                        pl.multiple_of(lo_r + _NS * rb, 8),
                                        rem)])
            plsc.subcore_barrier()

    return sk


def _sort_pad_edges(recv, send, inv, nr, ns, nch):
    e = recv.shape[0]
    recv = recv.astype(_i32)
    perm = jnp.argsort(recv)
    recv_s = recv[perm]
    send_s = send.astype(_i32)[perm]
    inv_s = inv[perm]
    r = nr // nch
    off = jnp.searchsorted(recv_s, jnp.arange(nch + 1, dtype=_i32) * r)
    off = jnp.concatenate(
        [off.astype(_i32), jnp.zeros((64 - nch - 1,), _i32)])
    epad = ((e + _EGRAN - 1) // _EGRAN) * _EGRAN
    pad = epad - e
    fill = jnp.arange(pad, dtype=_i32)
    recv_p = jnp.concatenate([recv_s, fill % nr])
    send_p = jnp.concatenate([send_s, fill % ns])
    inv_p = jnp.concatenate([inv_s, jnp.zeros((pad, inv.shape[1]), _f32)])
    return (recv_p.reshape(epad // 128, 128), send_p.reshape(epad // 128, 128),
            inv_p, off, e, epad)


def _split_msg(mp, ni):
    w1 = mp["l1"]["W"]
    return {"w1r": w1[:_HID], "w1s": w1[_HID:2 * _HID], "w1i": w1[2 * _HID:],
            "b1": mp["l1"]["b"], "w2": mp["l2"]["W"], "b2": mp["l2"]["b"]}


def kernel(x_0, x_1, x_2, adj_0, adj_1, inc_0_1, inc_1_2, inv_0_0, inv_1_1,
           inv_0_1, inv_1_2, batch_0, batch_1, batch_2, params):
    p00 = _sort_pad_edges(adj_0[1], adj_0[0], inv_0_0, _N0, _N0, 2)
    p11 = _sort_pad_edges(adj_1[1], adj_1[0], inv_1_1, _N1, _N1, 20)
    p01 = _sort_pad_edges(inc_0_1[1], inc_0_1[0], inv_0_1, _N1, _N0, 20)
    p12 = _sort_pad_edges(inc_1_2[1], inc_1_2[0], inv_1_2, _N2, _N1, 20)
    zeros = jnp.zeros((_ZR, _HID), _f32)

    emb = params["emb"]
    h0 = _mm_bias(x_0, emb["W"], emb["b"])
    h1 = _mm_bias(x_1, emb["W"], emb["b"])
    h2 = _mm_bias(x_2, emb["W"], emb["b"])

    for lp in params["layers"]:
        sp = {k: _split_msg(lp[k], ni) for k, ni in
              (("msg_adj_0", 3), ("msg_adj_1", 6),
               ("msg_inc_0_1", 3), ("msg_inc_1_2", 6))}
        zb = jnp.zeros((_HID,), _f32)
        t0a, t0b = _stage1(
            h0,
            jnp.stack([sp["msg_adj_0"]["w1r"], sp["msg_adj_0"]["w1s"],
                       sp["msg_inc_0_1"]["w1s"], sp["msg_inc_0_1"]["w1s"]]),
            jnp.stack([sp["msg_adj_0"]["b1"], zb, zb, zb]).reshape(4, 1, _HID))
        t1, t1b = _stage1(
            h1,
            jnp.stack([sp["msg_adj_1"]["w1r"], sp["msg_adj_1"]["w1s"],
                       sp["msg_inc_0_1"]["w1r"], sp["msg_inc_1_2"]["w1s"]]),
            jnp.stack([sp["msg_adj_1"]["b1"], zb, sp["msg_inc_0_1"]["b1"],
                       zb]).reshape(4, 1, _HID))
        (t2,) = _stage1(
            h2,
            jnp.stack([sp["msg_inc_1_2"]["w1r"], sp["msg_inc_1_2"]["w1r"]]),
            jnp.stack([sp["msg_inc_1_2"]["b1"],
                       sp["msg_inc_1_2"]["b1"]]).reshape(2, 1, _HID))

        msums = []
        for (name, atab, btab, pe, nr, nch) in (
                ("msg_adj_0", t0a, t0a, p00, _N0, 2),
                ("msg_adj_1", t1, t1, p11, _N1, 20),
                ("msg_inc_0_1", t1b, t0b, p01, _N1, 20),
                ("msg_inc_1_2", t2, t1b, p12, _N2, 20)):
            recv2, send2, inv_p, off, e, epad = pe
            tsum = _make_gather(epad)(atab, btab, recv2, send2)
            m = _edge_mlp(tsum, inv_p, sp[name]["w1i"], sp[name]["w2"],
                          sp[name]["b2"], e)
            msums.append(_make_scatter(epad, nr, nch)(m, recv2, off, zeros))

        h0 = _update(h0, [msums[0]], lp["upd_0"]["W"], lp["upd_0"]["b"])
        h1 = _update(h1, [msums[1], msums[2]], lp["upd_1"]["W"], lp["upd_1"]["b"])
        h2 = _update(h2, [msums[3]], lp["upd_2"]["W"], lp["upd_2"]["b"])

    pools = [_pool(h0, batch_0, params["pre"]["rank_0"]),
             _pool(h1, batch_1, params["pre"]["rank_1"]),
             _pool(h2, batch_2, params["pre"]["rank_2"])]
    state = jnp.concatenate(pools, axis=1)
    post = params["post"]
    out = pl.pallas_call(
        _post_body,
        out_shape=jax.ShapeDtypeStruct((_NG, 1), _f32),
    )(state, post["l1"]["W"], post["l1"]["b"].reshape(1, _HID),
      post["l2"]["W"], post["l2"]["b"].reshape(1, 1))
    return jnp.squeeze(out)
